# Initial kernel scaffold; baseline (speedup 1.0000x reference)
import jax
import jax.numpy as jnp
from jax import lax
from jax.experimental import pallas as pl
from jax.experimental.pallas import tpu as pltpu
from jax.experimental.pallas import tpu_sc as plsc

F = 26
B = 16384
VOCAB = 100000
NC, NS = 2, 16
NW = NC * NS
BPW = B // NW
EPW = BPW * F
GCH = 128
NG = EPW // GCH
NBUF = 8


def _body(x_hbm, table_hbm, bias_hbm, out_hbm, xv, vals, acc, biasv, sem):
    wid = lax.axis_index("s") * NC + lax.axis_index("c")
    iota = lax.iota(jnp.int32, 16)

    pltpu.sync_copy(x_hbm.at[pl.ds(wid * NG, NG)], xv)
    pltpu.sync_copy(bias_hbm, biasv)

    def add_off(r, _):
        def add_off_c(cc, _):
            pos = r * GCH + cc * 16 + iota
            off = lax.rem(pos, F) * VOCAB
            xv[r, pl.ds(cc * 16, 16)] = xv[r, pl.ds(cc * 16, 16)] + off
            return 0

        lax.fori_loop(0, GCH // 16, add_off_c, 0)
        return 0

    lax.fori_loop(0, NG, add_off, 0)

    def fire(g):
        pltpu.async_copy(table_hbm.at[xv.at[g]], vals.at[g], sem)

    def drain(g):
        pltpu.make_async_copy(table_hbm.at[xv.at[g]], vals.at[g], sem).wait()

    for g in range(NBUF):
        fire(g)

    def gloop(g, _):
        @pl.when(g + NBUF < NG)
        def _():
            fire(g + NBUF)

        drain(g)
        return 0

    lax.fori_loop(0, NG, gloop, 0)

    bv = biasv[...]

    def red(c, _):
        base = c * (16 * F) + iota * F

        def redf(f, v):
            idx = base + f
            return v + plsc.load_gather(
                vals,
                [lax.shift_right_logical(idx, 7), lax.bitwise_and(idx, 127)],
            )

        v = lax.fori_loop(0, F, redf, bv)
        acc[pl.ds(c * 16, 16)] = v
        return 0

    lax.fori_loop(0, BPW // 16, red, 0)

    pltpu.sync_copy(acc, out_hbm.at[pl.ds(wid * BPW, BPW)])


def kernel(x, table, bias):
    x_flat = x.reshape(NW * NG, GCH)
    t_flat = table.reshape(-1)
    b16 = jnp.broadcast_to(bias, (16,))
    run = pl.kernel(
        _body,
        mesh=plsc.VectorSubcoreMesh(core_axis_name="c", subcore_axis_name="s"),
        out_type=jax.ShapeDtypeStruct((B,), jnp.float32),
        scratch_types=[
            pltpu.VMEM((NG, GCH), jnp.int32),
            pltpu.VMEM((NG, GCH), jnp.float32),
            pltpu.VMEM((BPW,), jnp.float32),
            pltpu.VMEM((16,), jnp.float32),
            pltpu.SemaphoreType.DMA,
        ],
    )
    return run(x_flat, t_flat, b16).reshape(B, 1)


# speedup vs baseline: 1.1708x; 1.1708x over previous
import jax
import jax.numpy as jnp
from jax import lax
from jax.experimental import pallas as pl
from jax.experimental.pallas import tpu as pltpu
from jax.experimental.pallas import tpu_sc as plsc

F = 26
B = 16384
VOCAB = 100000
NC, NS = 2, 16
NW = NC * NS
BPW = B // NW
EPW = BPW * F
GCH = 128
NG = EPW // GCH
NBUF = 8


def _body(x_hbm, table_hbm, bias_hbm, out_hbm, xv, vals, acc, biasv, sem):
    wid = lax.axis_index("s") * NC + lax.axis_index("c")

    pltpu.sync_copy(x_hbm.at[pl.ds(wid * NG, NG)], xv)
    pltpu.sync_copy(bias_hbm, biasv)

    def add_off(r, _):
        off = lax.shift_right_logical(r, 2) * VOCAB

        def add_off_c(cc, _):
            xv[r, pl.ds(cc * 16, 16)] = xv[r, pl.ds(cc * 16, 16)] + off
            return 0

        lax.fori_loop(0, GCH // 16, add_off_c, 0)
        return 0

    lax.fori_loop(0, NG, add_off, 0)

    def fire(g):
        pltpu.async_copy(table_hbm.at[xv.at[g]], vals.at[pl.ds(g * GCH, GCH)], sem)

    def drain(g):
        pltpu.make_async_copy(
            table_hbm.at[xv.at[g]], vals.at[pl.ds(g * GCH, GCH)], sem
        ).wait()

    for g in range(NBUF):
        fire(g)

    def gloop(g, _):
        @pl.when(g + NBUF < NG)
        def _():
            fire(g + NBUF)

        drain(g)
        return 0

    lax.fori_loop(0, NG, gloop, 0)

    bv = biasv[...]

    def red(c, _):
        def redf(f, v):
            return v + vals[pl.ds(f * BPW + c * 16, 16)]

        v = lax.fori_loop(0, F, redf, bv)
        acc[pl.ds(c * 16, 16)] = v
        return 0

    lax.fori_loop(0, BPW // 16, red, 0)

    pltpu.sync_copy(acc, out_hbm.at[pl.ds(wid * BPW, BPW)])


def kernel(x, table, bias):
    x_flat = (
        x.T.reshape(F, NW, BPW).transpose(1, 0, 2).reshape(NW * NG, GCH)
    )
    t_flat = table.reshape(-1)
    b16 = jnp.broadcast_to(bias, (16,))
    run = pl.kernel(
        _body,
        mesh=plsc.VectorSubcoreMesh(core_axis_name="c", subcore_axis_name="s"),
        out_type=jax.ShapeDtypeStruct((B,), jnp.float32),
        scratch_types=[
            pltpu.VMEM((NG, GCH), jnp.int32),
            pltpu.VMEM((EPW,), jnp.float32),
            pltpu.VMEM((BPW,), jnp.float32),
            pltpu.VMEM((16,), jnp.float32),
            pltpu.SemaphoreType.DMA,
        ],
    )
    return run(x_flat, t_flat, b16).reshape(B, 1)


# speedup vs baseline: 1.1739x; 1.0026x over previous
import jax
import jax.numpy as jnp
from jax import lax
from jax.experimental import pallas as pl
from jax.experimental.pallas import tpu as pltpu
from jax.experimental.pallas import tpu_sc as plsc

F = 26
B = 16384
VOCAB = 100000
NC, NS = 2, 16
NW = NC * NS
BPW = B // NW
EPW = BPW * F
GCH = 128
NG = EPW // GCH
NBUF = 8


def _body(x_hbm, table_hbm, bias_hbm, out_hbm, xv, vals, acc, biasv, sem):
    wid = lax.axis_index("s") * NC + lax.axis_index("c")

    pltpu.sync_copy(x_hbm.at[pl.ds(wid * NG, NG)], xv)
    pltpu.sync_copy(bias_hbm, biasv)

    def fire(g):
        pltpu.async_copy(table_hbm.at[xv.at[g]], vals.at[pl.ds(g * GCH, GCH)], sem)

    def drain(g):
        pltpu.make_async_copy(
            table_hbm.at[xv.at[g]], vals.at[pl.ds(g * GCH, GCH)], sem
        ).wait()

    for g in range(NBUF):
        fire(g)

    def gloop(g, _):
        @pl.when(g + NBUF < NG)
        def _():
            fire(g + NBUF)

        drain(g)
        return 0

    lax.fori_loop(0, NG, gloop, 0)

    bv = biasv[...]

    def red(c, _):
        def redf(f, v):
            return v + vals[pl.ds(f * BPW + c * 16, 16)]

        v = lax.fori_loop(0, F, redf, bv)
        acc[pl.ds(c * 16, 16)] = v
        return 0

    lax.fori_loop(0, BPW // 16, red, 0)

    pltpu.sync_copy(acc, out_hbm.at[pl.ds(wid * BPW, BPW)])


def kernel(x, table, bias):
    offs = (jnp.arange(F, dtype=jnp.int32) * VOCAB)[None, None, :]
    x_flat = (
        (x.reshape(NW, BPW, F) + offs)
        .transpose(0, 2, 1)
        .reshape(NW * NG, GCH)
    )
    t_flat = table[:, 0]
    b16 = jnp.broadcast_to(bias, (16,))
    run = pl.kernel(
        _body,
        mesh=plsc.VectorSubcoreMesh(core_axis_name="c", subcore_axis_name="s"),
        out_type=jax.ShapeDtypeStruct((B,), jnp.float32),
        scratch_types=[
            pltpu.VMEM((NG, GCH), jnp.int32),
            pltpu.VMEM((EPW,), jnp.float32),
            pltpu.VMEM((BPW,), jnp.float32),
            pltpu.VMEM((16,), jnp.float32),
            pltpu.SemaphoreType.DMA,
        ],
    )
    return run(x_flat, t_flat, b16).reshape(B, 1)
